# baseline (speedup 1.0000x reference)
import functools

import jax
import jax.numpy as jnp
from jax import lax
from jax.experimental import pallas as pl
from jax.experimental.pallas import tpu as pltpu
from jax.experimental.pallas import tpu_sc as plsc

N, E, D, H, L, G = 10000, 320000, 128, 128, 3, 64

NC, NS = 2, 16
CH = 128
KB = 16
NCH0 = 160
NBLK0 = NCH0 // KB
TOTCH = NS * NCH0
EPAD = TOTCH * CH
NP = 10112
STRIPE = NP // NS

_SC_MESH = plsc.VectorSubcoreMesh(
    core_axis_name="c", subcore_axis_name="s", num_cores=NC, num_subcores=NS
)


@functools.partial(
    pl.kernel,
    out_type=jax.ShapeDtypeStruct((NP, H), jnp.float32),
    mesh=_SC_MESH,
    scratch_types=[
        pltpu.VMEM((KB, CH), jnp.int32),
        pltpu.VMEM((KB, CH), jnp.int32),
        pltpu.VMEM((KB, CH), jnp.int32),
        pltpu.VMEM((KB, CH), jnp.int32),
        pltpu.VMEM((CH, H), jnp.float32),
        pltpu.VMEM((CH, H), jnp.float32),
        pltpu.VMEM_SHARED((NP, H), jnp.float32),
        pltpu.SemaphoreType.DMA,
        pltpu.SemaphoreType.DMA,
        pltpu.SemaphoreType.DMA,
        pltpu.SemaphoreType.DMA,
        pltpu.SemaphoreType.DMA,
    ],
)
def _segsum(h_hbm, src_hbm, dst_hbm, zero_hbm, out_hbm,
            src_a, src_b, dst_a, dst_b, rows_a, rows_b, agg_sh,
            gsem_a, gsem_b, ssem_a, ssem_b, isem):
    c = lax.axis_index("c")
    s = lax.axis_index("s")

    srcbufs = (src_a, src_b)
    dstbufs = (dst_a, dst_b)
    rows = (rows_a, rows_b)
    gsems = (gsem_a, gsem_b)
    ssems = (ssem_a, ssem_b)

    def run(nblk, rowbase):
        pltpu.sync_copy(src_hbm.at[pl.ds(rowbase, KB)], src_a)
        pltpu.sync_copy(dst_hbm.at[pl.ds(rowbase, KB)], dst_a)
        pltpu.async_copy(h_hbm.at[src_a.at[0]], rows_a, gsem_a)

        def pairblock(mm, acc):
            for mb in range(2):
                m = 2 * mm + mb
                sv, dv = srcbufs[mb], dstbufs[mb]
                nsv, ndv = srcbufs[1 - mb], dstbufs[1 - mb]

                def pair(k, acc2):
                    for b in range(2):
                        r = 2 * k + b
                        j = m * KB + r
                        cur, nxt = rows[b], rows[1 - b]
                        pltpu.make_async_copy(h_hbm.at[sv.at[0]], cur,
                                              gsems[b]).wait()

                        @pl.when(j > 0)
                        def _():
                            pltpu.make_async_copy(nxt, agg_sh.at[dv.at[0]],
                                                  ssems[1 - b]).wait()

                        @pl.when(jnp.logical_and(r == 0, m + 1 < nblk))
                        def _():
                            base2 = rowbase + (m + 1) * KB
                            pltpu.async_copy(src_hbm.at[pl.ds(base2, KB)],
                                             nsv, isem)
                            pltpu.async_copy(dst_hbm.at[pl.ds(base2, KB)],
                                             ndv, isem)

                        @pl.when(r < KB - 1)
                        def _():
                            pltpu.async_copy(h_hbm.at[sv.at[r + 1]], nxt,
                                             gsems[1 - b])

                        @pl.when(jnp.logical_and(r == KB - 1, m + 1 < nblk))
                        def _():
                            pltpu.make_async_copy(src_hbm.at[pl.ds(0, KB)],
                                                  nsv, isem).wait()
                            pltpu.make_async_copy(dst_hbm.at[pl.ds(0, KB)],
                                                  ndv, isem).wait()
                            pltpu.async_copy(h_hbm.at[nsv.at[0]], nxt,
                                             gsems[1 - b])

                        pltpu.async_copy(cur, agg_sh.at[dv.at[r]], ssems[b],
                                         add=True)
                    return acc2

                lax.fori_loop(0, KB // 2, pair, 0)
            return acc

        lax.fori_loop(0, nblk // 2, pairblock, 0)
        pltpu.make_async_copy(rows_b, agg_sh.at[dst_a.at[0]], ssem_b).wait()

    @pl.when(c == 1)
    def _():
        pltpu.sync_copy(zero_hbm.at[pl.ds(s * STRIPE, STRIPE)],
                        agg_sh.at[pl.ds(s * STRIPE, STRIPE)])
        plsc.subcore_barrier()
        run(NBLK0, s * NCH0)
        plsc.subcore_barrier()
        pltpu.sync_copy(agg_sh.at[pl.ds(s * STRIPE, STRIPE)],
                        out_hbm.at[pl.ds(s * STRIPE, STRIPE)])


_BLK = 1000
_GRID = N // _BLK


def _layer_body(scale_ref, h_ref, agg_ref, w1_ref, b1_ref, w2_ref, b2_ref,
                out_ref, cs_ref):
    i = pl.program_id(0)
    z = h_ref[...] * scale_ref[0, 0] + agg_ref[...]
    y = jnp.maximum(
        jnp.dot(z, w1_ref[...], preferred_element_type=jnp.float32) + b1_ref[...], 0.0)
    hn = jnp.dot(y, w2_ref[...], preferred_element_type=jnp.float32) + b2_ref[...]
    out_ref[...] = hn

    @pl.when(i == 0)
    def _():
        cs_ref[...] = jnp.zeros_like(cs_ref)

    cs_ref[0:1, :] += jnp.sum(hn, axis=0, keepdims=True)


def _layer(h, aggp, scale, W1, b1, W2, b2):
    return pl.pallas_call(
        _layer_body,
        grid=(_GRID,),
        in_specs=[
            pl.BlockSpec((1, 1), lambda i: (0, 0), memory_space=pltpu.SMEM),
            pl.BlockSpec((_BLK, H), lambda i: (i, 0)),
            pl.BlockSpec((_BLK, H), lambda i: (i, 0)),
            pl.BlockSpec((D, H), lambda i: (0, 0)),
            pl.BlockSpec((1, H), lambda i: (0, 0)),
            pl.BlockSpec((H, H), lambda i: (0, 0)),
            pl.BlockSpec((1, H), lambda i: (0, 0)),
        ],
        out_specs=[
            pl.BlockSpec((_BLK, H), lambda i: (i, 0)),
            pl.BlockSpec((8, H), lambda i: (0, 0)),
        ],
        out_shape=[
            jax.ShapeDtypeStruct((N, H), jnp.float32),
            jax.ShapeDtypeStruct((8, H), jnp.float32),
        ],
    )(scale, h, aggp, W1, b1, W2, b2)


def _head_body(c1_ref, c2_ref, c3_ref, wl1_ref, bl1_ref, wl2_ref, bl2_ref, out_ref):
    o = (jnp.dot(c1_ref[0:1, :], wl1_ref[0:H, :], preferred_element_type=jnp.float32)
         + jnp.dot(c2_ref[0:1, :], wl1_ref[H:2 * H, :], preferred_element_type=jnp.float32)
         + jnp.dot(c3_ref[0:1, :], wl1_ref[2 * H:3 * H, :], preferred_element_type=jnp.float32)
         + bl1_ref[...])
    o = jnp.maximum(o, 0.0)
    o2 = jnp.maximum(
        jnp.dot(o, wl2_ref[...], preferred_element_type=jnp.float32) + bl2_ref[...], 0.0)
    out_ref[...] = jnp.broadcast_to(o2, (8, H))


def _head(c1, c2, c3, Wl1, bl1, Wl2p, bl2p):
    return pl.pallas_call(
        _head_body,
        out_shape=jax.ShapeDtypeStruct((8, H), jnp.float32),
    )(c1, c2, c3, Wl1, bl1, Wl2p, bl2p)


def kernel(x, edge_index, edge_attr, batch, edge_batch, Ws1, bs1, Ws2, bs2, eps,
           Wl1, bl1, Wl2, bl2):
    src = edge_index[0]
    dst = edge_index[1]
    pad = EPAD - E
    srcp = jnp.concatenate([src, jnp.zeros((pad,), jnp.int32)]).reshape(TOTCH, CH)
    dstp = jnp.concatenate([dst, jnp.full((pad,), N, jnp.int32)]).reshape(TOTCH, CH)
    zero_np = jnp.zeros((NP, H), jnp.float32)

    h = x
    csums = []
    for l in range(L):
        aggp = _segsum(h, srcp, dstp, zero_np)
        scale = (1.0 + eps[l]).reshape(1, 1)
        h, cs = _layer(h, aggp, scale, Ws1[l], bs1[l].reshape(1, H),
                       Ws2[l], bs2[l].reshape(1, H))
        csums.append(cs)

    Wl2p = jnp.zeros((H, H), jnp.float32).at[:, :64].set(Wl2)
    bl2p = jnp.zeros((1, H), jnp.float32).at[0, :64].set(bl2)
    out = _head(csums[0], csums[1], csums[2], Wl1, bl1.reshape(1, H), Wl2p, bl2p)
    return out[0:1, 0:64]


# speedup vs baseline: 1.4575x; 1.4575x over previous
import functools

import jax
import jax.numpy as jnp
from jax import lax
from jax.experimental import pallas as pl
from jax.experimental.pallas import tpu as pltpu
from jax.experimental.pallas import tpu_sc as plsc

N, E, D, H, L, G = 10000, 320000, 128, 128, 3, 64

NC, NS = 2, 16
CH = 128
KB = 8
NCH0 = 144
NCH1 = 16
NBLK0 = NCH0 // KB
NBLK1 = NCH1 // KB
TOTCH = NS * (NCH0 + NCH1)
EPAD = TOTCH * CH
NP = 10112
STRIPE = NP // NS
ZROWS = 40

_SC_MESH = plsc.VectorSubcoreMesh(
    core_axis_name="c", subcore_axis_name="s", num_cores=NC, num_subcores=NS
)


@functools.partial(
    pl.kernel,
    out_type=jax.ShapeDtypeStruct((NC, NP, H), jnp.float32),
    mesh=_SC_MESH,
    scratch_types=[
        pltpu.VMEM((KB, CH), jnp.int32),
        pltpu.VMEM((KB, CH), jnp.int32),
        pltpu.VMEM((KB, CH), jnp.int32),
        pltpu.VMEM((KB, CH), jnp.int32),
        pltpu.VMEM((CH, H), jnp.float32),
        pltpu.VMEM((CH, H), jnp.float32),
        pltpu.VMEM((ZROWS, H), jnp.float32),
        pltpu.VMEM_SHARED((NP, H), jnp.float32),
        pltpu.SemaphoreType.DMA,
        pltpu.SemaphoreType.DMA,
        pltpu.SemaphoreType.DMA,
        pltpu.SemaphoreType.DMA,
        pltpu.SemaphoreType.DMA,
    ],
)
def _segsum(h_hbm, src_hbm, dst_hbm, out_hbm,
            src_a, src_b, dst_a, dst_b, rows_a, rows_b, zbuf, agg_sh,
            gsem_a, gsem_b, ssem_a, ssem_b, isem):
    c = lax.axis_index("c")
    s = lax.axis_index("s")

    def zrow(i, acc):
        for t in range(H // 16):
            zbuf[i, pl.ds(t * 16, 16)] = jnp.zeros((16,), jnp.float32)
        return acc

    lax.fori_loop(0, ZROWS, zrow, 0)

    def zcopy(m, acc):
        pltpu.sync_copy(zbuf, agg_sh.at[pl.ds(s * STRIPE + m * ZROWS, ZROWS)])
        return acc

    lax.fori_loop(0, STRIPE // ZROWS, zcopy, 0)
    pltpu.sync_copy(zbuf.at[pl.ds(0, STRIPE % ZROWS)],
                    agg_sh.at[pl.ds(s * STRIPE + (STRIPE // ZROWS) * ZROWS,
                                    STRIPE % ZROWS)])
    plsc.subcore_barrier()

    srcbufs = (src_a, src_b)
    dstbufs = (dst_a, dst_b)
    rows = (rows_a, rows_b)
    gsems = (gsem_a, gsem_b)
    ssems = (ssem_a, ssem_b)

    def run(nblk, rowbase):
        pltpu.sync_copy(src_hbm.at[pl.ds(rowbase, KB)], src_a)
        pltpu.sync_copy(dst_hbm.at[pl.ds(rowbase, KB)], dst_a)
        pltpu.async_copy(h_hbm.at[src_a.at[0]], rows_a, gsem_a)

        def pairblock(mm, acc):
            for mb in range(2):
                m = 2 * mm + mb
                sv, dv = srcbufs[mb], dstbufs[mb]
                nsv, ndv = srcbufs[1 - mb], dstbufs[1 - mb]

                def pair(k, acc2):
                    for b in range(2):
                        r = 2 * k + b
                        j = m * KB + r
                        cur, nxt = rows[b], rows[1 - b]
                        pltpu.make_async_copy(h_hbm.at[sv.at[0]], cur,
                                              gsems[b]).wait()

                        @pl.when(j > 0)
                        def _():
                            pltpu.make_async_copy(nxt, agg_sh.at[dv.at[0]],
                                                  ssems[1 - b]).wait()

                        @pl.when(jnp.logical_and(r == 0, m + 1 < nblk))
                        def _():
                            base2 = rowbase + (m + 1) * KB
                            pltpu.async_copy(src_hbm.at[pl.ds(base2, KB)],
                                             nsv, isem)
                            pltpu.async_copy(dst_hbm.at[pl.ds(base2, KB)],
                                             ndv, isem)

                        @pl.when(r < KB - 1)
                        def _():
                            pltpu.async_copy(h_hbm.at[sv.at[r + 1]], nxt,
                                             gsems[1 - b])

                        @pl.when(jnp.logical_and(r == KB - 1, m + 1 < nblk))
                        def _():
                            pltpu.make_async_copy(src_hbm.at[pl.ds(0, KB)],
                                                  nsv, isem).wait()
                            pltpu.make_async_copy(dst_hbm.at[pl.ds(0, KB)],
                                                  ndv, isem).wait()
                            pltpu.async_copy(h_hbm.at[nsv.at[0]], nxt,
                                             gsems[1 - b])

                        pltpu.async_copy(cur, agg_sh.at[dv.at[r]], ssems[b],
                                         add=True)
                    return acc2

                lax.fori_loop(0, KB // 2, pair, 0)
            return acc

        lax.fori_loop(0, nblk // 2, pairblock, 0)
        pltpu.make_async_copy(rows_b, agg_sh.at[dst_a.at[0]], ssem_b).wait()

    @pl.when(c == 0)
    def _():
        run(NBLK0, s * NCH0)

    @pl.when(c == 1)
    def _():
        run(NBLK1, NS * NCH0 + s * NCH1)

    plsc.subcore_barrier()
    pltpu.sync_copy(agg_sh.at[pl.ds(s * STRIPE, STRIPE)],
                    out_hbm.at[c, pl.ds(s * STRIPE, STRIPE)])


_BLK = 1000
_GRID = N // _BLK


def _layer_body(scale_ref, h_ref, agg_ref, w1_ref, b1_ref, w2_ref, b2_ref,
                out_ref, cs_ref):
    i = pl.program_id(0)
    z = h_ref[...] * scale_ref[0, 0] + agg_ref[0] + agg_ref[1]
    y = jnp.maximum(
        jnp.dot(z, w1_ref[...], preferred_element_type=jnp.float32) + b1_ref[...], 0.0)
    hn = jnp.dot(y, w2_ref[...], preferred_element_type=jnp.float32) + b2_ref[...]
    out_ref[...] = hn

    @pl.when(i == 0)
    def _():
        cs_ref[...] = jnp.zeros_like(cs_ref)

    cs_ref[0:1, :] += jnp.sum(hn, axis=0, keepdims=True)


def _layer(h, aggp, scale, W1, b1, W2, b2):
    return pl.pallas_call(
        _layer_body,
        grid=(_GRID,),
        in_specs=[
            pl.BlockSpec((1, 1), lambda i: (0, 0), memory_space=pltpu.SMEM),
            pl.BlockSpec((_BLK, H), lambda i: (i, 0)),
            pl.BlockSpec((NC, _BLK, H), lambda i: (0, i, 0)),
            pl.BlockSpec((D, H), lambda i: (0, 0)),
            pl.BlockSpec((1, H), lambda i: (0, 0)),
            pl.BlockSpec((H, H), lambda i: (0, 0)),
            pl.BlockSpec((1, H), lambda i: (0, 0)),
        ],
        out_specs=[
            pl.BlockSpec((_BLK, H), lambda i: (i, 0)),
            pl.BlockSpec((8, H), lambda i: (0, 0)),
        ],
        out_shape=[
            jax.ShapeDtypeStruct((N, H), jnp.float32),
            jax.ShapeDtypeStruct((8, H), jnp.float32),
        ],
    )(scale, h, aggp, W1, b1, W2, b2)


def _head_body(c1_ref, c2_ref, c3_ref, wl1_ref, bl1_ref, wl2_ref, bl2_ref, out_ref):
    o = (jnp.dot(c1_ref[0:1, :], wl1_ref[0:H, :], preferred_element_type=jnp.float32)
         + jnp.dot(c2_ref[0:1, :], wl1_ref[H:2 * H, :], preferred_element_type=jnp.float32)
         + jnp.dot(c3_ref[0:1, :], wl1_ref[2 * H:3 * H, :], preferred_element_type=jnp.float32)
         + bl1_ref[...])
    o = jnp.maximum(o, 0.0)
    o2 = jnp.maximum(
        jnp.dot(o, wl2_ref[...], preferred_element_type=jnp.float32) + bl2_ref[...], 0.0)
    out_ref[...] = jnp.broadcast_to(o2, (8, H))


def _head(c1, c2, c3, Wl1, bl1, Wl2p, bl2p):
    return pl.pallas_call(
        _head_body,
        out_shape=jax.ShapeDtypeStruct((8, H), jnp.float32),
    )(c1, c2, c3, Wl1, bl1, Wl2p, bl2p)


def kernel(x, edge_index, edge_attr, batch, edge_batch, Ws1, bs1, Ws2, bs2, eps,
           Wl1, bl1, Wl2, bl2):
    src = edge_index[0]
    dst = edge_index[1]
    pad = EPAD - E
    srcp = jnp.concatenate([src, jnp.zeros((pad,), jnp.int32)]).reshape(TOTCH, CH)
    dstp = jnp.concatenate([dst, jnp.full((pad,), N, jnp.int32)]).reshape(TOTCH, CH)

    h = x
    csums = []
    for l in range(L):
        aggp = _segsum(h, srcp, dstp)
        scale = (1.0 + eps[l]).reshape(1, 1)
        h, cs = _layer(h, aggp, scale, Ws1[l], bs1[l].reshape(1, H),
                       Ws2[l], bs2[l].reshape(1, H))
        csums.append(cs)

    Wl2p = jnp.zeros((H, H), jnp.float32).at[:, :64].set(Wl2)
    bl2p = jnp.zeros((1, H), jnp.float32).at[0, :64].set(bl2)
    out = _head(csums[0], csums[1], csums[2], Wl1, bl1.reshape(1, H), Wl2p, bl2p)
    return out[0:1, 0:64]
